# baseline (speedup 1.0000x reference)
import functools

import jax
import jax.numpy as jnp
from jax.experimental import pallas as pl
from jax.experimental.pallas import tpu as pltpu
from jax.experimental.pallas import tpu_sc as plsc

HID = 1024
MAXL = 8192
BATCH = 4

NC, NS, L = 2, 16, 16
NW = NC * NS
ROWS_PER_W = MAXL // NW
CHUNK = 32
NCH = ROWS_PER_W // CHUNK
RWIN = CHUNK + 16

S = 512
NB = MAXL // S


def _sc_interp_body(vel_hbm, pe_hbm, out_hbm, vel_v, slab_v, out_v, sem):
    wid = jax.lax.axis_index("s") * NC + jax.lax.axis_index("c")
    w0 = wid * ROWS_PER_W

    pltpu.sync_copy(vel_hbm, vel_v)
    ig_s = vel_v[...][0]

    for ch in range(NCH):
        g0 = w0 + ch * CHUNK
        g0_f = g0.astype(jnp.float32)
        base_f = g0_f * ig_s
        base_i = base_f.astype(jnp.int32) - 8
        base_i = jnp.clip(base_i, 0, MAXL - RWIN)
        base_i = (base_i // 8) * 8
        base_i = pl.multiple_of(base_i, 8)
        pltpu.sync_copy(pe_hbm.at[pl.ds(base_i, RWIN), :], slab_v)

        def row_body(r, carry):
            g_f = g0_f + r.astype(jnp.float32)
            rel = jnp.clip(g_f * ig_s, 0.0, float(MAXL - 1))
            lo = rel.astype(jnp.int32)
            lo = lo - (lo.astype(jnp.float32) > rel).astype(jnp.int32)
            wh = rel - lo.astype(jnp.float32)
            wl = 1.0 - wh
            ll = jnp.clip(lo - base_i, 0, RWIN - 2)

            def col_body(c, carry2):
                a = slab_v[ll, pl.ds(c * L, L)]
                b = slab_v[ll + 1, pl.ds(c * L, L)]
                out_v[r, pl.ds(c * L, L)] = a * wl + b * wh
                return carry2

            jax.lax.fori_loop(0, HID // L, col_body, 0, unroll=4)
            return carry

        jax.lax.fori_loop(0, CHUNK, row_body, 0)
        pltpu.sync_copy(out_v, out_hbm.at[pl.ds(g0, CHUNK), :])


def _sc_interp(vel16, pe2d):
    mesh = plsc.VectorSubcoreMesh(core_axis_name="c", subcore_axis_name="s")
    kern = functools.partial(
        pl.kernel,
        mesh=mesh,
        out_type=jax.ShapeDtypeStruct((MAXL, HID), jnp.float32),
        scratch_types=[
            pltpu.VMEM((L,), jnp.float32),
            pltpu.VMEM((RWIN, HID), jnp.float32),
            pltpu.VMEM((CHUNK, HID), jnp.float32),
            pltpu.SemaphoreType.DMA,
        ],
    )(_sc_interp_body)
    return kern(vel16, pe2d)


def _tc_add_kernel(x_ref, pe_ref, o_ref):
    o_ref[...] = x_ref[...] + pe_ref[...][None, :, :]


def _tc_add(x, pe_interp):
    return pl.pallas_call(
        _tc_add_kernel,
        grid=(NB,),
        in_specs=[
            pl.BlockSpec((BATCH, S, HID), lambda i: (0, i, 0)),
            pl.BlockSpec((S, HID), lambda i: (i, 0)),
        ],
        out_specs=pl.BlockSpec((BATCH, S, HID), lambda i: (0, i, 0)),
        out_shape=jax.ShapeDtypeStruct(x.shape, x.dtype),
    )(x, pe_interp)


def kernel(x, velocity, pe_base):
    pe2d = pe_base[0]
    v = jnp.clip(velocity, 0.0, 0.99)
    inv_gamma16 = jnp.broadcast_to(jnp.sqrt(1.0 - v * v), (L,)).astype(jnp.float32)
    pe_interp = _sc_interp(inv_gamma16, pe2d)
    return _tc_add(x, pe_interp)


# speedup vs baseline: 1.4261x; 1.4261x over previous
import jax
import jax.numpy as jnp
from jax.experimental import pallas as pl
from jax.experimental.pallas import tpu as pltpu

HID = 1024
MAXL = 8192
BATCH = 4
S = 512
NB = MAXL // S
R = S + 16


def _pe_add_kernel(b_ref, vel_ref, pe_hbm, x_ref, o_ref, slab_ref, sem_ref):
    i = pl.program_id(0)

    def start_copy(step, slot):
        pltpu.make_async_copy(
            pe_hbm.at[pl.ds(pl.multiple_of(b_ref[step], 8), R), :],
            slab_ref.at[slot], sem_ref.at[slot]).start()

    @pl.when(i == 0)
    def _():
        start_copy(0, 0)

    @pl.when(i + 1 < NB)
    def _():
        start_copy(i + 1, (i + 1) % 2)

    base = b_ref[i]
    pltpu.make_async_copy(
        pe_hbm.at[pl.ds(pl.multiple_of(base, 8), R), :],
        slab_ref.at[i % 2], sem_ref.at[i % 2]).wait()

    v = jnp.clip(vel_ref[0, 0], 0.0, 0.99)
    gamma = 1.0 / jnp.sqrt(1.0 - v * v)
    pos = (jax.lax.broadcasted_iota(jnp.int32, (S, 1), 0) + i * S).astype(jnp.float32)
    rel = jnp.clip(pos / gamma, 0.0, float(MAXL - 1))
    rfl = jnp.floor(rel)
    wh = rel - rfl
    wl = 1.0 - wh
    lo = rfl.astype(jnp.int32) - base
    lo = jnp.clip(lo, 0, R - 1)
    hi = jnp.minimum(lo + 1, R - 1)
    col = jax.lax.broadcasted_iota(jnp.int32, (S, R), 1)
    w = jnp.where(col == lo, wl, 0.0) + jnp.where(col == hi, wh, 0.0)
    pe = jax.lax.dot_general(
        w.astype(jnp.bfloat16), slab_ref[i % 2].astype(jnp.bfloat16),
        (((1,), (0,)), ((), ())), preferred_element_type=jnp.float32)
    o_ref[...] = x_ref[...] + pe[None, :, :]


def kernel(x, velocity, pe_base):
    pe2d = pe_base[0]
    v = jnp.clip(velocity[0], 0.0, 0.99)
    gamma = 1.0 / jnp.sqrt(1.0 - v * v)
    p0 = jnp.arange(NB, dtype=jnp.float32) * S
    b = jnp.floor(jnp.clip(p0 / gamma, 0.0, float(MAXL - 1)))
    b_arr = jnp.clip(jnp.floor((b - 4.0) / 8.0) * 8.0, 0.0,
                     float(MAXL - R)).astype(jnp.int32)
    vel2d = velocity.reshape(1, 1)

    grid_spec = pltpu.PrefetchScalarGridSpec(
        num_scalar_prefetch=1,
        grid=(NB,),
        in_specs=[
            pl.BlockSpec((1, 1), lambda i, bb: (0, 0)),
            pl.BlockSpec(memory_space=pl.ANY),
            pl.BlockSpec((BATCH, S, HID), lambda i, bb: (0, i, 0)),
        ],
        out_specs=pl.BlockSpec((BATCH, S, HID), lambda i, bb: (0, i, 0)),
        scratch_shapes=[
            pltpu.VMEM((2, R, HID), jnp.float32),
            pltpu.SemaphoreType.DMA((2,)),
        ],
    )
    return pl.pallas_call(
        _pe_add_kernel,
        grid_spec=grid_spec,
        out_shape=jax.ShapeDtypeStruct(x.shape, x.dtype),
        compiler_params=pltpu.CompilerParams(
            dimension_semantics=("arbitrary",)),
        input_output_aliases={3: 0},
    )(b_arr, vel2d, pe2d, x)


# speedup vs baseline: 2.2189x; 1.5559x over previous
import jax
import jax.numpy as jnp
from jax.experimental import pallas as pl
from jax.experimental.pallas import tpu as pltpu

HID = 1024
MAXL = 8192
BATCH = 4
S = 512
NB = MAXL // S
R = S + 16


def _pe_add_kernel(b_ref, vel_ref, pe_hbm, x_ref, o_ref, slab_ref, pe_ref,
                   sem_ref):
    i = pl.program_id(0)
    bi = pl.program_id(1)

    def start_copy(step, slot):
        pltpu.make_async_copy(
            pe_hbm.at[pl.ds(pl.multiple_of(b_ref[step], 8), R), :],
            slab_ref.at[slot], sem_ref.at[slot]).start()

    @pl.when((i == 0) & (bi == 0))
    def _():
        start_copy(0, 0)

    @pl.when((bi == 0) & (i + 1 < NB))
    def _():
        start_copy(i + 1, (i + 1) % 2)

    @pl.when(bi == 0)
    def _():
        base = b_ref[i]
        pltpu.make_async_copy(
            pe_hbm.at[pl.ds(pl.multiple_of(base, 8), R), :],
            slab_ref.at[i % 2], sem_ref.at[i % 2]).wait()

        v = jnp.clip(vel_ref[0, 0], 0.0, 0.99)
        gamma = 1.0 / jnp.sqrt(1.0 - v * v)
        pos = (jax.lax.broadcasted_iota(jnp.int32, (S, 1), 0)
               + i * S).astype(jnp.float32)
        rel = jnp.clip(pos / gamma, 0.0, float(MAXL - 1))
        rfl = jnp.floor(rel)
        wh = rel - rfl
        wl = 1.0 - wh
        lo = rfl.astype(jnp.int32) - base
        lo = jnp.clip(lo, 0, R - 1)
        hi = jnp.minimum(lo + 1, R - 1)
        col = jax.lax.broadcasted_iota(jnp.int32, (S, R), 1)
        w = jnp.where(col == lo, wl, 0.0) + jnp.where(col == hi, wh, 0.0)
        pe_ref[...] = jax.lax.dot_general(
            w.astype(jnp.bfloat16), slab_ref[i % 2].astype(jnp.bfloat16),
            (((1,), (0,)), ((), ())), preferred_element_type=jnp.float32)

    o_ref[...] = x_ref[...] + pe_ref[...][None, :, :]


def kernel(x, velocity, pe_base):
    pe2d = pe_base[0]
    v = jnp.clip(velocity[0], 0.0, 0.99)
    gamma = 1.0 / jnp.sqrt(1.0 - v * v)
    p0 = jnp.arange(NB, dtype=jnp.float32) * S
    b = jnp.floor(jnp.clip(p0 / gamma, 0.0, float(MAXL - 1)))
    b_arr = jnp.clip(jnp.floor((b - 4.0) / 8.0) * 8.0, 0.0,
                     float(MAXL - R)).astype(jnp.int32)
    vel2d = velocity.reshape(1, 1)

    grid_spec = pltpu.PrefetchScalarGridSpec(
        num_scalar_prefetch=1,
        grid=(NB, BATCH),
        in_specs=[
            pl.BlockSpec((1, 1), lambda i, bi, bb: (0, 0)),
            pl.BlockSpec(memory_space=pl.ANY),
            pl.BlockSpec((1, S, HID), lambda i, bi, bb: (bi, i, 0)),
        ],
        out_specs=pl.BlockSpec((1, S, HID), lambda i, bi, bb: (bi, i, 0)),
        scratch_shapes=[
            pltpu.VMEM((2, R, HID), jnp.float32),
            pltpu.VMEM((S, HID), jnp.float32),
            pltpu.SemaphoreType.DMA((2,)),
        ],
    )
    return pl.pallas_call(
        _pe_add_kernel,
        grid_spec=grid_spec,
        out_shape=jax.ShapeDtypeStruct(x.shape, x.dtype),
        compiler_params=pltpu.CompilerParams(
            dimension_semantics=("arbitrary", "arbitrary")),
    )(b_arr, vel2d, pe2d, x)


# speedup vs baseline: 2.6488x; 1.1937x over previous
import jax
import jax.numpy as jnp
from jax.experimental import pallas as pl
from jax.experimental.pallas import tpu as pltpu

HID = 1024
MAXL = 8192
BATCH = 4
S = 256
NB = MAXL // S
R = S + 16


def _pe_add_kernel(b_ref, vel_ref, pe_hbm, x_ref, o_ref, slab_ref, sem_ref):
    i = pl.program_id(0)

    def start_copy(step, slot):
        pltpu.make_async_copy(
            pe_hbm.at[pl.ds(pl.multiple_of(b_ref[step], 8), R), :],
            slab_ref.at[slot], sem_ref.at[slot]).start()

    @pl.when(i == 0)
    def _():
        start_copy(0, 0)

    @pl.when(i + 1 < NB)
    def _():
        start_copy(i + 1, (i + 1) % 2)

    base = b_ref[i]
    pltpu.make_async_copy(
        pe_hbm.at[pl.ds(pl.multiple_of(base, 8), R), :],
        slab_ref.at[i % 2], sem_ref.at[i % 2]).wait()

    v = jnp.clip(vel_ref[0, 0], 0.0, 0.99)
    gamma = 1.0 / jnp.sqrt(1.0 - v * v)
    pos = (jax.lax.broadcasted_iota(jnp.int32, (S, 1), 0) + i * S).astype(jnp.float32)
    rel = jnp.clip(pos / gamma, 0.0, float(MAXL - 1))
    rfl = jnp.floor(rel)
    wh = rel - rfl
    wl = 1.0 - wh
    lo = rfl.astype(jnp.int32) - base
    lo = jnp.clip(lo, 0, R - 1)
    hi = jnp.minimum(lo + 1, R - 1)
    col = jax.lax.broadcasted_iota(jnp.int32, (S, R), 1)
    w = jnp.where(col == lo, wl, 0.0) + jnp.where(col == hi, wh, 0.0)
    pe = jax.lax.dot_general(
        w.astype(jnp.bfloat16), slab_ref[i % 2].astype(jnp.bfloat16),
        (((1,), (0,)), ((), ())), preferred_element_type=jnp.float32)
    o_ref[...] = x_ref[...] + pe[None, :, :]


def kernel(x, velocity, pe_base):
    pe2d = pe_base[0]
    v = jnp.clip(velocity[0], 0.0, 0.99)
    gamma = 1.0 / jnp.sqrt(1.0 - v * v)
    p0 = jnp.arange(NB, dtype=jnp.float32) * S
    b = jnp.floor(jnp.clip(p0 / gamma, 0.0, float(MAXL - 1)))
    b_arr = jnp.clip(jnp.floor((b - 4.0) / 8.0) * 8.0, 0.0,
                     float(MAXL - R)).astype(jnp.int32)
    vel2d = velocity.reshape(1, 1)

    grid_spec = pltpu.PrefetchScalarGridSpec(
        num_scalar_prefetch=1,
        grid=(NB,),
        in_specs=[
            pl.BlockSpec((1, 1), lambda i, bb: (0, 0)),
            pl.BlockSpec(memory_space=pl.ANY),
            pl.BlockSpec((BATCH, S, HID), lambda i, bb: (0, i, 0)),
        ],
        out_specs=pl.BlockSpec((BATCH, S, HID), lambda i, bb: (0, i, 0)),
        scratch_shapes=[
            pltpu.VMEM((2, R, HID), jnp.float32),
            pltpu.SemaphoreType.DMA((2,)),
        ],
    )
    return pl.pallas_call(
        _pe_add_kernel,
        grid_spec=grid_spec,
        out_shape=jax.ShapeDtypeStruct(x.shape, x.dtype),
        compiler_params=pltpu.CompilerParams(
            dimension_semantics=("arbitrary",)),
    )(b_arr, vel2d, pe2d, x)
